# baseline (speedup 1.0000x reference)
import functools

import jax
import jax.numpy as jnp
from jax import lax
from jax.experimental import pallas as pl
from jax.experimental.pallas import tpu as pltpu
from jax.experimental.pallas import tpu_sc as plsc

N = 10000
E = 160000
G = 64
C = 128
NT = 16
EPT = E // NT
B = 128
EPT_PAD = 10240
NB = EPT_PAD // B
NBH = NB // 2
EH = NBH * B
PAD = EPT_PAD - EPT
ROWS_PAD = 10240
ZPT = ROWS_PAD // NT
CPT = 632
CPT_LAST = N - 15 * CPT


@functools.cache
def _sc_agg(J: int):
    Jh = J // 2
    mesh = plsc.VectorSubcoreMesh(core_axis_name="c", subcore_axis_name="s")

    @functools.partial(
        pl.kernel,
        out_type=jax.ShapeDtypeStruct((N, J * C), jnp.float32),
        mesh=mesh,
        scratch_types=[
            pltpu.VMEM((EH,), jnp.int32),
            pltpu.VMEM((2 * NBH, B // 2), jnp.int32),
            pltpu.VMEM((B, C), jnp.float32),
            pltpu.VMEM((B, C), jnp.float32),
            pltpu.VMEM_SHARED((ROWS_PAD, C), jnp.float32),
            pltpu.SemaphoreType.DMA,
            pltpu.SemaphoreType.DMA,
            pltpu.SemaphoreType.DMA,
            pltpu.SemaphoreType.DMA,
        ],
    )
    def agg_kernel(x2, src1, dst3, zeros, out, idx_v, dst_v, rows0_v,
                   rows1_v, agg_sp, sem0, sem1, sem2, sem3):
        c = lax.axis_index("c")
        s = lax.axis_index("s")
        H = B // 2

        def _start(bi, rows_v, sem):
            pltpu.async_copy(
                x2.at[idx_v.at[pl.ds(bi * B, B)]], rows_v, sem)

        def _wait(bi, rows_v, sem):
            pltpu.make_async_copy(
                x2.at[idx_v.at[pl.ds(bi * B, B)]], rows_v, sem).wait()

        def _scat_start(bi, rows_v):
            pltpu.async_copy(rows_v.at[pl.ds(0, H)],
                             agg_sp.at[dst_v.at[2 * bi]], sem2, add=True)
            pltpu.async_copy(rows_v.at[pl.ds(H, H)],
                             agg_sp.at[dst_v.at[2 * bi + 1]], sem3, add=True)

        def _scat_wait(bi, rows_v):
            pltpu.make_async_copy(rows_v.at[pl.ds(0, H)],
                                  agg_sp.at[dst_v.at[2 * bi]], sem2).wait()
            pltpu.make_async_copy(rows_v.at[pl.ds(H, H)],
                                  agg_sp.at[dst_v.at[2 * bi + 1]], sem3).wait()

        def _run_half(h, j):
            pltpu.sync_copy(
                src1.at[pl.ds(s * EPT_PAD + h * EH, EH)], idx_v)
            pltpu.sync_copy(dst3.at[s, pl.ds(h * 2 * NBH, 2 * NBH)], dst_v)

            def ibody(i, carry):
                sl = pl.ds(i * 16, 16)
                idx_v[sl] = idx_v[sl] * J + j
                return carry
            lax.fori_loop(0, EH // 16, ibody, 0)

        def _edge_loop():
            _start(0, rows0_v, sem0)
            _wait(0, rows0_v, sem0)

            def ebody(i, carry):
                b0 = 2 * i
                b1 = b0 + 1
                _start(b1, rows1_v, sem1)
                _scat_start(b0, rows0_v)
                _wait(b1, rows1_v, sem1)
                _scat_wait(b0, rows0_v)
                b2 = jnp.minimum(b0 + 2, NBH - 1)
                _start(b2, rows0_v, sem0)
                _scat_start(b1, rows1_v)
                _wait(b2, rows0_v, sem0)
                _scat_wait(b1, rows1_v)
                return carry
            lax.fori_loop(0, NBH // 2, ebody, 0)

        for jh in range(Jh):
            j = c * Jh + jh
            pltpu.sync_copy(zeros, agg_sp.at[pl.ds(s * ZPT, ZPT)])
            _run_half(0, j)
            plsc.subcore_barrier()
            _edge_loop()
            _run_half(1, j)
            _edge_loop()

            plsc.subcore_barrier()

            @pl.when(s < NT - 1)
            def _copy_main():
                pltpu.sync_copy(
                    agg_sp.at[pl.ds(s * CPT, CPT)],
                    out.at[pl.ds(s * CPT, CPT), pl.ds(j * C, C)])

            @pl.when(s == NT - 1)
            def _copy_last():
                pltpu.sync_copy(
                    agg_sp.at[pl.ds(s * CPT, CPT_LAST)],
                    out.at[pl.ds(s * CPT, CPT_LAST), pl.ds(j * C, C)])

            plsc.subcore_barrier()

    return agg_kernel


def _aggregate(h, src_p, dst_p, zeros):
    J = h.shape[1] // C
    return _sc_agg(J)(h.reshape(N * J, C), src_p, dst_p, zeros)


@functools.cache
def _mm(din: int, dout: int, BM: int = 1000, BN: int = 1024, BK: int = 512):
    grid = (N // BM, dout // BN, din // BK)

    def kfn(a_ref, x_ref, wr_ref, ws_ref, b_ref, o_ref):
        k = pl.program_id(2)
        nk = pl.num_programs(2)
        part = jnp.dot(a_ref[...], wr_ref[...],
                       preferred_element_type=jnp.float32)
        part = part + jnp.dot(x_ref[...], ws_ref[...],
                              preferred_element_type=jnp.float32)

        @pl.when(k == 0)
        def _init():
            o_ref[...] = part

        @pl.when(k > 0)
        def _acc():
            o_ref[...] = o_ref[...] + part

        @pl.when(k == nk - 1)
        def _fin():
            o_ref[...] = jnp.maximum(o_ref[...] + b_ref[...], 0.0)

    return pl.pallas_call(
        kfn,
        grid=grid,
        in_specs=[
            pl.BlockSpec((BM, BK), lambda m, n, k: (m, k)),
            pl.BlockSpec((BM, BK), lambda m, n, k: (m, k)),
            pl.BlockSpec((BK, BN), lambda m, n, k: (k, n)),
            pl.BlockSpec((BK, BN), lambda m, n, k: (k, n)),
            pl.BlockSpec((1, BN), lambda m, n, k: (0, n)),
        ],
        out_specs=pl.BlockSpec((BM, BN), lambda m, n, k: (m, n)),
        out_shape=jax.ShapeDtypeStruct((N, dout), jnp.float32),
    )


def _graph_conv(h, src_p, dst_p, zeros, Wr, b, Ws):
    din = h.shape[1]
    dout = Wr.shape[0]
    agg = _aggregate(h, src_p, dst_p, zeros)
    return _mm(din, dout)(agg, h, Wr.T, Ws.T, b.reshape(1, dout))


BMP = 400

@functools.cache
def _pool(D: int):
    grid = (N // BMP,)

    def kfn(bs_ref, bv_ref, h_ref, o_ref):
        m = pl.program_id(0)

        @pl.when(m == 0)
        def _init():
            o_ref[...] = jnp.full_like(o_ref, -jnp.inf)

        g0 = bs_ref[0, 0, 0]
        g1 = bs_ref[0, 0, BMP - 1]
        bv = bv_ref[0]
        h = h_ref[...]

        def body(g, carry):
            mask = bv == g
            v = jnp.max(jnp.where(mask, h, -jnp.inf), axis=0, keepdims=True)
            o_ref[pl.ds(g, 1), :] = jnp.maximum(o_ref[pl.ds(g, 1), :], v)
            return carry

        lax.fori_loop(g0, g1 + 1, body, 0)

    return pl.pallas_call(
        kfn,
        grid=grid,
        in_specs=[
            pl.BlockSpec((1, 1, BMP), lambda m: (m, 0, 0),
                         memory_space=pltpu.SMEM),
            pl.BlockSpec((1, BMP, 1), lambda m: (m, 0, 0)),
            pl.BlockSpec((BMP, D), lambda m: (m, 0)),
        ],
        out_specs=pl.BlockSpec((G, D), lambda m: (0, 0)),
        out_shape=jax.ShapeDtypeStruct((G, D), jnp.float32),
    )


@functools.cache
def _dense(D: int, H: int, O: int):
    def kfn(p_ref, w1_ref, b1_ref, w2_ref, b2_ref, o_ref):
        t = jnp.dot(p_ref[...], w1_ref[...],
                    preferred_element_type=jnp.float32) + b1_ref[...]
        t = jnp.maximum(t, 0.0)
        o_ref[...] = jnp.dot(t, w2_ref[...],
                             preferred_element_type=jnp.float32) + b2_ref[...]

    return pl.pallas_call(
        kfn,
        out_shape=jax.ShapeDtypeStruct((G, O), jnp.float32),
    )


def kernel(x, edge_index, batch, W1r, b1, W1s, W2r, b2, W2s, W3r, b3, W3s,
           W4r, b4, W4s, Wf1, bf1, Wf2, bf2):
    src = edge_index[0]
    dst = edge_index[1]

    fill_s = jnp.broadcast_to(
        (jnp.arange(PAD, dtype=jnp.int32) * 83) % N, (NT, PAD))
    fill_d = jnp.broadcast_to(
        N + jnp.arange(PAD, dtype=jnp.int32), (NT, PAD))
    src_p = jnp.concatenate(
        [src.reshape(NT, EPT), fill_s], axis=1).reshape(NT * EPT_PAD)
    dst_p = jnp.concatenate(
        [dst.reshape(NT, EPT), fill_d], axis=1).reshape(NT, 2 * NB, B // 2)
    zeros = jnp.zeros((ZPT, C), jnp.float32)

    h = jnp.pad(x, ((0, 0), (0, 124)))
    W1r_p = jnp.pad(W1r, ((0, 0), (0, 124)))
    W1s_p = jnp.pad(W1s, ((0, 0), (0, 124)))

    h = _graph_conv(h, src_p, dst_p, zeros, W1r_p, b1, W1s_p)
    h = _graph_conv(h, src_p, dst_p, zeros, W2r, b2, W2s)
    h = _graph_conv(h, src_p, dst_p, zeros, W3r, b3, W3s)
    h = _graph_conv(h, src_p, dst_p, zeros, W4r, b4, W4s)

    D = h.shape[1]
    batch3 = batch.reshape(N // BMP, 1, BMP)
    batchc = batch.reshape(N // BMP, BMP, 1)
    pooled = _pool(D)(batch3, batchc, h)

    return _dense(D, Wf1.shape[0], Wf2.shape[0])(
        pooled, Wf1.T, bf1.reshape(1, -1), Wf2.T, bf2.reshape(1, -1))


# speedup vs baseline: 1.0045x; 1.0045x over previous
import functools

import jax
import jax.numpy as jnp
from jax import lax
from jax.experimental import pallas as pl
from jax.experimental.pallas import tpu as pltpu
from jax.experimental.pallas import tpu_sc as plsc

N = 10000
E = 160000
G = 64
C = 128
NT = 16
EPT = E // NT
B = 128
EPT_PAD = 10240
NB = EPT_PAD // B
NBH = NB // 2
EH = NBH * B
PAD = EPT_PAD - EPT
ROWS_PAD = 10240
ZPT = ROWS_PAD // NT
CPT = 632
CPT_LAST = N - 15 * CPT


@functools.cache
def _sc_agg(J: int):
    Jh = J // 2
    mesh = plsc.VectorSubcoreMesh(core_axis_name="c", subcore_axis_name="s")

    @functools.partial(
        pl.kernel,
        out_type=jax.ShapeDtypeStruct((N, J * C), jnp.float32),
        mesh=mesh,
        scratch_types=[
            pltpu.VMEM((EH,), jnp.int32),
            pltpu.VMEM((2 * NBH, B // 2), jnp.int32),
            pltpu.VMEM((B, C), jnp.float32),
            pltpu.VMEM((B, C), jnp.float32),
            pltpu.VMEM_SHARED((ROWS_PAD, C), jnp.float32),
            pltpu.SemaphoreType.DMA,
            pltpu.SemaphoreType.DMA,
            pltpu.SemaphoreType.DMA,
            pltpu.SemaphoreType.DMA,
        ],
    )
    def agg_kernel(x2, src1, dst3, zeros, out, idx_v, dst_v, rows0_v,
                   rows1_v, agg_sp, sem0, sem1, sem2, sem3):
        c = lax.axis_index("c")
        s = lax.axis_index("s")
        H = B // 2

        def _start(bi, rows_v, sem):
            pltpu.async_copy(
                x2.at[idx_v.at[pl.ds(bi * B, B)]], rows_v, sem)

        def _wait(bi, rows_v, sem):
            pltpu.make_async_copy(
                x2.at[idx_v.at[pl.ds(bi * B, B)]], rows_v, sem).wait()

        def _scat_start(bi, rows_v):
            pltpu.async_copy(rows_v.at[pl.ds(0, H)],
                             agg_sp.at[dst_v.at[2 * bi]], sem2, add=True)
            pltpu.async_copy(rows_v.at[pl.ds(H, H)],
                             agg_sp.at[dst_v.at[2 * bi + 1]], sem3, add=True)

        def _scat_wait(bi, rows_v):
            pltpu.make_async_copy(rows_v.at[pl.ds(0, H)],
                                  agg_sp.at[dst_v.at[2 * bi]], sem2).wait()
            pltpu.make_async_copy(rows_v.at[pl.ds(H, H)],
                                  agg_sp.at[dst_v.at[2 * bi + 1]], sem3).wait()

        def _run_half(h, j):
            pltpu.sync_copy(
                src1.at[pl.ds(s * EPT_PAD + h * EH, EH)], idx_v)
            pltpu.sync_copy(dst3.at[s, pl.ds(h * 2 * NBH, 2 * NBH)], dst_v)

            def ibody(i, carry):
                sl = pl.ds(i * 16, 16)
                idx_v[sl] = idx_v[sl] * J + j
                return carry
            lax.fori_loop(0, EH // 16, ibody, 0)

        def _edge_loop():
            _start(0, rows0_v, sem0)
            _wait(0, rows0_v, sem0)

            def ebody(i, carry):
                b0 = 2 * i
                b1 = b0 + 1
                _start(b1, rows1_v, sem1)
                _scat_start(b0, rows0_v)
                _wait(b1, rows1_v, sem1)
                _scat_wait(b0, rows0_v)
                b2 = jnp.minimum(b0 + 2, NBH - 1)
                _start(b2, rows0_v, sem0)
                _scat_start(b1, rows1_v)
                _wait(b2, rows0_v, sem0)
                _scat_wait(b1, rows1_v)
                return carry
            lax.fori_loop(0, NBH // 2, ebody, 0)

        for jh in range(Jh):
            j = c * Jh + jh
            pltpu.sync_copy(zeros, agg_sp.at[pl.ds(s * ZPT, ZPT)])
            _run_half(0, j)
            plsc.subcore_barrier()
            _edge_loop()
            _run_half(1, j)
            _edge_loop()

            plsc.subcore_barrier()

            @pl.when(s < NT - 1)
            def _copy_main():
                pltpu.sync_copy(
                    agg_sp.at[pl.ds(s * CPT, CPT)],
                    out.at[pl.ds(s * CPT, CPT), pl.ds(j * C, C)])

            @pl.when(s == NT - 1)
            def _copy_last():
                pltpu.sync_copy(
                    agg_sp.at[pl.ds(s * CPT, CPT_LAST)],
                    out.at[pl.ds(s * CPT, CPT_LAST), pl.ds(j * C, C)])

            plsc.subcore_barrier()

    return agg_kernel


def _aggregate(h, src_p, dst_p, zeros):
    J = h.shape[1] // C
    return _sc_agg(J)(h.reshape(N * J, C), src_p, dst_p, zeros)


@functools.cache
def _mm_x(din: int, dout: int, BM: int = 1000, BN: int = 1024, BK: int = 512):
    grid = (N // BM, dout // BN, din // BK)

    def kfn(x_ref, ws_ref, b_ref, o_ref):
        k = pl.program_id(2)
        part = jnp.dot(x_ref[...], ws_ref[...],
                       preferred_element_type=jnp.float32)

        @pl.when(k == 0)
        def _init():
            o_ref[...] = part + b_ref[...]

        @pl.when(k > 0)
        def _acc():
            o_ref[...] = o_ref[...] + part

    return pl.pallas_call(
        kfn,
        grid=grid,
        in_specs=[
            pl.BlockSpec((BM, BK), lambda m, n, k: (m, k)),
            pl.BlockSpec((BK, BN), lambda m, n, k: (k, n)),
            pl.BlockSpec((1, BN), lambda m, n, k: (0, n)),
        ],
        out_specs=pl.BlockSpec((BM, BN), lambda m, n, k: (m, n)),
        out_shape=jax.ShapeDtypeStruct((N, dout), jnp.float32),
    )


@functools.cache
def _mm_a(din: int, dout: int, BM: int = 1000, BN: int = 1024, BK: int = 512):
    grid = (N // BM, dout // BN, din // BK)

    def kfn(a_ref, wr_ref, p_ref, o_ref):
        k = pl.program_id(2)
        nk = pl.num_programs(2)
        part = jnp.dot(a_ref[...], wr_ref[...],
                       preferred_element_type=jnp.float32)

        @pl.when(k == 0)
        def _init():
            o_ref[...] = part + p_ref[...]

        @pl.when((k > 0) & (k < nk - 1))
        def _acc():
            o_ref[...] = o_ref[...] + part

        @pl.when((k == nk - 1) & (k > 0))
        def _fin():
            o_ref[...] = jnp.maximum(o_ref[...] + part, 0.0)

        @pl.when(nk == 1)
        def _fin1():
            o_ref[...] = jnp.maximum(o_ref[...], 0.0)

    return pl.pallas_call(
        kfn,
        grid=grid,
        in_specs=[
            pl.BlockSpec((BM, BK), lambda m, n, k: (m, k)),
            pl.BlockSpec((BK, BN), lambda m, n, k: (k, n)),
            pl.BlockSpec((BM, BN), lambda m, n, k: (m, n)),
        ],
        out_specs=pl.BlockSpec((BM, BN), lambda m, n, k: (m, n)),
        out_shape=jax.ShapeDtypeStruct((N, dout), jnp.float32),
    )


def _graph_conv(h, src_p, dst_p, zeros, Wr, b, Ws):
    din = h.shape[1]
    dout = Wr.shape[0]
    agg = _aggregate(h, src_p, dst_p, zeros)
    xs = _mm_x(din, dout)(h, Ws.T, b.reshape(1, dout))
    return _mm_a(din, dout)(agg, Wr.T, xs)


BMP = 400

@functools.cache
def _pool(D: int):
    grid = (N // BMP,)

    def kfn(bs_ref, bv_ref, h_ref, o_ref):
        m = pl.program_id(0)

        @pl.when(m == 0)
        def _init():
            o_ref[...] = jnp.full_like(o_ref, -jnp.inf)

        g0 = bs_ref[0, 0, 0]
        g1 = bs_ref[0, 0, BMP - 1]
        bv = bv_ref[0]
        h = h_ref[...]

        def body(g, carry):
            mask = bv == g
            v = jnp.max(jnp.where(mask, h, -jnp.inf), axis=0, keepdims=True)
            o_ref[pl.ds(g, 1), :] = jnp.maximum(o_ref[pl.ds(g, 1), :], v)
            return carry

        lax.fori_loop(g0, g1 + 1, body, 0)

    return pl.pallas_call(
        kfn,
        grid=grid,
        in_specs=[
            pl.BlockSpec((1, 1, BMP), lambda m: (m, 0, 0),
                         memory_space=pltpu.SMEM),
            pl.BlockSpec((1, BMP, 1), lambda m: (m, 0, 0)),
            pl.BlockSpec((BMP, D), lambda m: (m, 0)),
        ],
        out_specs=pl.BlockSpec((G, D), lambda m: (0, 0)),
        out_shape=jax.ShapeDtypeStruct((G, D), jnp.float32),
    )


@functools.cache
def _dense(D: int, H: int, O: int):
    def kfn(p_ref, w1_ref, b1_ref, w2_ref, b2_ref, o_ref):
        t = jnp.dot(p_ref[...], w1_ref[...],
                    preferred_element_type=jnp.float32) + b1_ref[...]
        t = jnp.maximum(t, 0.0)
        o_ref[...] = jnp.dot(t, w2_ref[...],
                             preferred_element_type=jnp.float32) + b2_ref[...]

    return pl.pallas_call(
        kfn,
        out_shape=jax.ShapeDtypeStruct((G, O), jnp.float32),
    )


def kernel(x, edge_index, batch, W1r, b1, W1s, W2r, b2, W2s, W3r, b3, W3s,
           W4r, b4, W4s, Wf1, bf1, Wf2, bf2):
    src = edge_index[0]
    dst = edge_index[1]

    fill_s = jnp.broadcast_to(
        (jnp.arange(PAD, dtype=jnp.int32) * 83) % N, (NT, PAD))
    fill_d = jnp.broadcast_to(
        N + jnp.arange(PAD, dtype=jnp.int32), (NT, PAD))
    src_p = jnp.concatenate(
        [src.reshape(NT, EPT), fill_s], axis=1).reshape(NT * EPT_PAD)
    dst_p = jnp.concatenate(
        [dst.reshape(NT, EPT), fill_d], axis=1).reshape(NT, 2 * NB, B // 2)
    zeros = jnp.zeros((ZPT, C), jnp.float32)

    h = jnp.pad(x, ((0, 0), (0, 124)))
    W1r_p = jnp.pad(W1r, ((0, 0), (0, 124)))
    W1s_p = jnp.pad(W1s, ((0, 0), (0, 124)))

    h = _graph_conv(h, src_p, dst_p, zeros, W1r_p, b1, W1s_p)
    h = _graph_conv(h, src_p, dst_p, zeros, W2r, b2, W2s)
    h = _graph_conv(h, src_p, dst_p, zeros, W3r, b3, W3s)
    h = _graph_conv(h, src_p, dst_p, zeros, W4r, b4, W4s)

    D = h.shape[1]
    batch3 = batch.reshape(N // BMP, 1, BMP)
    batchc = batch.reshape(N // BMP, BMP, 1)
    pooled = _pool(D)(batch3, batchc, h)

    return _dense(D, Wf1.shape[0], Wf2.shape[0])(
        pooled, Wf1.T, bf1.reshape(1, -1), Wf2.T, bf2.reshape(1, -1))


# speedup vs baseline: 1.0300x; 1.0254x over previous
import functools

import jax
import jax.numpy as jnp
from jax import lax
from jax.experimental import pallas as pl
from jax.experimental.pallas import tpu as pltpu
from jax.experimental.pallas import tpu_sc as plsc

N = 10000
E = 160000
G = 64
C = 128
NT = 16
EPT = E // NT
B = 128
EPT_PAD = 10240
NB = EPT_PAD // B
NBH = NB // 2
EH = NBH * B
PAD = EPT_PAD - EPT
ROWS_PAD = 10240
ZPT = ROWS_PAD // NT
CPT = 632
CPT_LAST = N - 15 * CPT


@functools.cache
def _sc_agg(J: int):
    Jh = J // 2
    mesh = plsc.VectorSubcoreMesh(core_axis_name="c", subcore_axis_name="s")

    @functools.partial(
        pl.kernel,
        out_type=jax.ShapeDtypeStruct((N, J * C), jnp.float32),
        mesh=mesh,
        scratch_types=[
            pltpu.VMEM((EH,), jnp.int32),
            pltpu.VMEM((2 * NBH, B // 2), jnp.int32),
            pltpu.VMEM((B, C), jnp.float32),
            pltpu.VMEM((B, C), jnp.float32),
            pltpu.VMEM_SHARED((ROWS_PAD, C), jnp.float32),
            pltpu.SemaphoreType.DMA,
            pltpu.SemaphoreType.DMA,
            pltpu.SemaphoreType.DMA,
            pltpu.SemaphoreType.DMA,
        ],
    )
    def agg_kernel(x2, src1, dst3, zeros, out, idx_v, dst_v, rows0_v,
                   rows1_v, agg_sp, sem0, sem1, sem2, sem3):
        c = lax.axis_index("c")
        s = lax.axis_index("s")
        H = B // 2

        def _start(bi, rows_v, sem):
            pltpu.async_copy(
                x2.at[idx_v.at[pl.ds(bi * B, B)]], rows_v, sem)

        def _wait(bi, rows_v, sem):
            pltpu.make_async_copy(
                x2.at[idx_v.at[pl.ds(bi * B, B)]], rows_v, sem).wait()

        def _scat_start(bi, rows_v):
            pltpu.async_copy(rows_v.at[pl.ds(0, H)],
                             agg_sp.at[dst_v.at[2 * bi]], sem2, add=True)
            pltpu.async_copy(rows_v.at[pl.ds(H, H)],
                             agg_sp.at[dst_v.at[2 * bi + 1]], sem3, add=True)

        def _scat_wait(bi, rows_v):
            pltpu.make_async_copy(rows_v.at[pl.ds(0, H)],
                                  agg_sp.at[dst_v.at[2 * bi]], sem2).wait()
            pltpu.make_async_copy(rows_v.at[pl.ds(H, H)],
                                  agg_sp.at[dst_v.at[2 * bi + 1]], sem3).wait()

        def _run_half(h, j):
            pltpu.sync_copy(
                src1.at[pl.ds(s * EPT_PAD + h * EH, EH)], idx_v)
            pltpu.sync_copy(dst3.at[s, pl.ds(h * 2 * NBH, 2 * NBH)], dst_v)

            def ibody(i, carry):
                sl = pl.ds(i * 16, 16)
                idx_v[sl] = idx_v[sl] * J + j
                return carry
            lax.fori_loop(0, EH // 16, ibody, 0)

        def _edge_loop():
            _start(0, rows0_v, sem0)
            _wait(0, rows0_v, sem0)

            def ebody(i, carry):
                b0 = 2 * i
                b1 = b0 + 1
                _start(b1, rows1_v, sem1)
                _scat_start(b0, rows0_v)
                _wait(b1, rows1_v, sem1)
                _scat_wait(b0, rows0_v)
                b2 = jnp.minimum(b0 + 2, NBH - 1)
                _start(b2, rows0_v, sem0)
                _scat_start(b1, rows1_v)
                _wait(b2, rows0_v, sem0)
                _scat_wait(b1, rows1_v)
                return carry
            lax.fori_loop(0, NBH // 2, ebody, 0)

        for jh in range(Jh):
            j = c * Jh + jh
            pltpu.sync_copy(zeros, agg_sp.at[pl.ds(s * ZPT, ZPT)])
            _run_half(0, j)
            plsc.subcore_barrier()
            _edge_loop()
            _run_half(1, j)
            _edge_loop()

            plsc.subcore_barrier()

            @pl.when(s < NT - 1)
            def _copy_main():
                pltpu.sync_copy(
                    agg_sp.at[pl.ds(s * CPT, CPT)],
                    out.at[pl.ds(s * CPT, CPT), pl.ds(j * C, C)])

            @pl.when(s == NT - 1)
            def _copy_last():
                pltpu.sync_copy(
                    agg_sp.at[pl.ds(s * CPT, CPT_LAST)],
                    out.at[pl.ds(s * CPT, CPT_LAST), pl.ds(j * C, C)])

            plsc.subcore_barrier()

    return agg_kernel


def _aggregate(h, src_p, dst_p, zeros):
    J = h.shape[1] // C
    return _sc_agg(J)(h.reshape(N * J, C), src_p, dst_p, zeros)


@functools.cache
def _mm_x(din: int, dout: int, BM: int = 1000, BN: int = 1024, BK: int = 1024):
    grid = (N // BM, dout // BN, din // BK)

    def kfn(x_ref, ws_ref, b_ref, o_ref):
        k = pl.program_id(2)
        part = jnp.dot(x_ref[...], ws_ref[...],
                       preferred_element_type=jnp.float32)

        @pl.when(k == 0)
        def _init():
            o_ref[...] = part + b_ref[...]

        @pl.when(k > 0)
        def _acc():
            o_ref[...] = o_ref[...] + part

    return pl.pallas_call(
        kfn,
        grid=grid,
        in_specs=[
            pl.BlockSpec((BM, BK), lambda m, n, k: (m, k)),
            pl.BlockSpec((BK, BN), lambda m, n, k: (k, n)),
            pl.BlockSpec((1, BN), lambda m, n, k: (0, n)),
        ],
        out_specs=pl.BlockSpec((BM, BN), lambda m, n, k: (m, n)),
        out_shape=jax.ShapeDtypeStruct((N, dout), jnp.float32),
    )


@functools.cache
def _mm_a(din: int, dout: int, BM: int = 1000, BN: int = 1024, BK: int = 1024):
    grid = (N // BM, dout // BN, din // BK)

    def kfn(a_ref, wr_ref, p_ref, o_ref):
        k = pl.program_id(2)
        nk = pl.num_programs(2)
        part = jnp.dot(a_ref[...], wr_ref[...],
                       preferred_element_type=jnp.float32)

        @pl.when(k == 0)
        def _init():
            o_ref[...] = part + p_ref[...]

        @pl.when((k > 0) & (k < nk - 1))
        def _acc():
            o_ref[...] = o_ref[...] + part

        @pl.when((k == nk - 1) & (k > 0))
        def _fin():
            o_ref[...] = jnp.maximum(o_ref[...] + part, 0.0)

        @pl.when(nk == 1)
        def _fin1():
            o_ref[...] = jnp.maximum(o_ref[...], 0.0)

    return pl.pallas_call(
        kfn,
        grid=grid,
        in_specs=[
            pl.BlockSpec((BM, BK), lambda m, n, k: (m, k)),
            pl.BlockSpec((BK, BN), lambda m, n, k: (k, n)),
            pl.BlockSpec((BM, BN), lambda m, n, k: (m, n)),
        ],
        out_specs=pl.BlockSpec((BM, BN), lambda m, n, k: (m, n)),
        out_shape=jax.ShapeDtypeStruct((N, dout), jnp.float32),
    )


def _graph_conv(h, src_p, dst_p, zeros, Wr, b, Ws):
    din = h.shape[1]
    dout = Wr.shape[0]
    agg = _aggregate(h, src_p, dst_p, zeros)
    xs = _mm_x(din, dout)(h, Ws.T, b.reshape(1, dout))
    return _mm_a(din, dout)(agg, Wr.T, xs)


BMP = 400

@functools.cache
def _pool(D: int):
    grid = (N // BMP,)

    def kfn(bs_ref, bv_ref, h_ref, o_ref):
        m = pl.program_id(0)

        @pl.when(m == 0)
        def _init():
            o_ref[...] = jnp.full_like(o_ref, -jnp.inf)

        g0 = bs_ref[0, 0, 0]
        g1 = bs_ref[0, 0, BMP - 1]
        bv = bv_ref[0]
        h = h_ref[...]

        def body(g, carry):
            mask = bv == g
            v = jnp.max(jnp.where(mask, h, -jnp.inf), axis=0, keepdims=True)
            o_ref[pl.ds(g, 1), :] = jnp.maximum(o_ref[pl.ds(g, 1), :], v)
            return carry

        lax.fori_loop(g0, g1 + 1, body, 0)

    return pl.pallas_call(
        kfn,
        grid=grid,
        in_specs=[
            pl.BlockSpec((1, 1, BMP), lambda m: (m, 0, 0),
                         memory_space=pltpu.SMEM),
            pl.BlockSpec((1, BMP, 1), lambda m: (m, 0, 0)),
            pl.BlockSpec((BMP, D), lambda m: (m, 0)),
        ],
        out_specs=pl.BlockSpec((G, D), lambda m: (0, 0)),
        out_shape=jax.ShapeDtypeStruct((G, D), jnp.float32),
    )


@functools.cache
def _dense(D: int, H: int, O: int):
    def kfn(p_ref, w1_ref, b1_ref, w2_ref, b2_ref, o_ref):
        t = jnp.dot(p_ref[...], w1_ref[...],
                    preferred_element_type=jnp.float32) + b1_ref[...]
        t = jnp.maximum(t, 0.0)
        o_ref[...] = jnp.dot(t, w2_ref[...],
                             preferred_element_type=jnp.float32) + b2_ref[...]

    return pl.pallas_call(
        kfn,
        out_shape=jax.ShapeDtypeStruct((G, O), jnp.float32),
    )


def kernel(x, edge_index, batch, W1r, b1, W1s, W2r, b2, W2s, W3r, b3, W3s,
           W4r, b4, W4s, Wf1, bf1, Wf2, bf2):
    src = edge_index[0]
    dst = edge_index[1]

    fill_s = jnp.broadcast_to(
        (jnp.arange(PAD, dtype=jnp.int32) * 83) % N, (NT, PAD))
    fill_d = jnp.broadcast_to(
        N + jnp.arange(PAD, dtype=jnp.int32), (NT, PAD))
    src_p = jnp.concatenate(
        [src.reshape(NT, EPT), fill_s], axis=1).reshape(NT * EPT_PAD)
    dst_p = jnp.concatenate(
        [dst.reshape(NT, EPT), fill_d], axis=1).reshape(NT, 2 * NB, B // 2)
    zeros = jnp.zeros((ZPT, C), jnp.float32)

    h = jnp.pad(x, ((0, 0), (0, 124)))
    W1r_p = jnp.pad(W1r, ((0, 0), (0, 124)))
    W1s_p = jnp.pad(W1s, ((0, 0), (0, 124)))

    h = _graph_conv(h, src_p, dst_p, zeros, W1r_p, b1, W1s_p)
    h = _graph_conv(h, src_p, dst_p, zeros, W2r, b2, W2s)
    h = _graph_conv(h, src_p, dst_p, zeros, W3r, b3, W3s)
    h = _graph_conv(h, src_p, dst_p, zeros, W4r, b4, W4s)

    D = h.shape[1]
    batch3 = batch.reshape(N // BMP, 1, BMP)
    batchc = batch.reshape(N // BMP, BMP, 1)
    pooled = _pool(D)(batch3, batchc, h)

    return _dense(D, Wf1.shape[0], Wf2.shape[0])(
        pooled, Wf1.T, bf1.reshape(1, -1), Wf2.T, bf2.reshape(1, -1))
